# Initial kernel scaffold; baseline (speedup 1.0000x reference)
import jax
import jax.numpy as jnp
from jax.experimental import pallas as pl


def kernel(answer, year, answer_table, yearly_table, question_table):
    raise NotImplementedError("write your pallas kernel here")


# speedup vs baseline: 4.6795x; 4.6795x over previous
import jax
import jax.numpy as jnp
from jax import lax
from jax.experimental import pallas as pl
from jax.experimental.pallas import tpu as pltpu
from jax.experimental.pallas import tpu_sc as plsc

B = 4096
Q = 100
D = 64
L = 16
NC = 2
NS = 16
NW = NC * NS
BPW = B // NW
NBUF_G = 4
NBUF_O = 2
C = D // L


def kernel(answer, year, answer_table, yearly_table, question_table):
    answer = answer.astype(jnp.int32)
    year_flat = year.reshape(-1).astype(jnp.int32)

    def body(answer_hbm, year_hbm, atab_hbm, ytab_hbm, qtab_hbm, out_hbm,
             idx_v, yidx_v, yr_v, qt_v, rows_v, stage_v,
             gsem, osem, ysem):
        wid = lax.axis_index("s") * NC + lax.axis_index("c")
        base = wid * BPW

        pltpu.sync_copy(answer_hbm.at[pl.ds(base, BPW)], idx_v)
        pltpu.sync_copy(year_hbm.at[pl.ds(base, BPW)], yidx_v)
        pltpu.sync_copy(qtab_hbm, qt_v)
        pltpu.async_copy(ytab_hbm.at[yidx_v], yr_v, ysem).wait()

        for j in range(NBUF_G):
            pltpu.async_copy(atab_hbm.at[idx_v.at[j]], rows_v.at[j],
                             gsem.at[j])

        def outer(o, carry):
            for j in range(NBUF_G):
                g = o * NBUF_G + j
                oj = j % NBUF_O
                pltpu.make_async_copy(
                    atab_hbm.at[idx_v.at[g]], rows_v.at[j], gsem.at[j]).wait()

                def _wait_out():
                    pltpu.make_async_copy(
                        stage_v.at[oj], out_hbm.at[base], osem.at[oj]).wait()
                if j >= NBUF_O:
                    _wait_out()
                else:
                    pl.when(o > 0)(_wait_out)

                yrow = tuple(yr_v[g, pl.ds(c * L, L)] for c in range(C))

                def qbody(q, ycarry):
                    for c in range(C):
                        stage_v[oj, q, pl.ds(c * L, L)] = (
                            rows_v[j, q, pl.ds(c * L, L)]
                            + qt_v[q, pl.ds(c * L, L)]
                            + ycarry[c])
                    return ycarry

                lax.fori_loop(0, Q, qbody, yrow)

                pltpu.async_copy(stage_v.at[oj], out_hbm.at[base + g],
                                 osem.at[oj])

                def _next_gather():
                    pltpu.async_copy(
                        atab_hbm.at[idx_v.at[g + NBUF_G]], rows_v.at[j],
                        gsem.at[j])
                pl.when(o < BPW // NBUF_G - 1)(_next_gather)
            return carry

        lax.fori_loop(0, BPW // NBUF_G, outer, 0)

        for oj in range(NBUF_O):
            pltpu.make_async_copy(
                stage_v.at[oj], out_hbm.at[base], osem.at[oj]).wait()

    mesh = plsc.VectorSubcoreMesh(core_axis_name="c", subcore_axis_name="s",
                                  num_cores=NC, num_subcores=NS)
    run = pl.kernel(
        body,
        out_type=jax.ShapeDtypeStruct((B, Q, D), jnp.float32),
        mesh=mesh,
        compiler_params=pltpu.CompilerParams(use_tc_tiling_on_sc=False),
        scratch_types=[
            pltpu.VMEM((BPW, Q), jnp.int32),
            pltpu.VMEM((BPW,), jnp.int32),
            pltpu.VMEM((BPW, D), jnp.float32),
            pltpu.VMEM((Q, D), jnp.float32),
            pltpu.VMEM((NBUF_G, Q, D), jnp.float32),
            pltpu.VMEM((NBUF_O, Q, D), jnp.float32),
            pltpu.SemaphoreType.DMA((NBUF_G,)),
            pltpu.SemaphoreType.DMA((NBUF_O,)),
            pltpu.SemaphoreType.DMA,
        ],
    )
    return run(answer, year_flat, answer_table, yearly_table, question_table)


# speedup vs baseline: 5.2383x; 1.1194x over previous
import jax
import jax.numpy as jnp
from jax import lax
from jax.experimental import pallas as pl
from jax.experimental.pallas import tpu as pltpu
from jax.experimental.pallas import tpu_sc as plsc

B = 4096
Q = 100
D = 64
DP = 128
L = 16
NC = 2
NS = 16
NW = NC * NS
BPW = B // NW
NBUF_G = 4
NBUF_O = 2
C = D // L


def kernel(answer, year, answer_table, yearly_table, question_table):
    answer = answer.astype(jnp.int32)
    year_flat = year.reshape(-1).astype(jnp.int32)
    atab_pad = jnp.pad(answer_table, ((0, 0), (0, DP - D)))
    ytab_pad = jnp.pad(yearly_table, ((0, 0), (0, DP - D)))

    def body(answer_hbm, year_hbm, atab_hbm, ytab_hbm, qtab_hbm, out_hbm,
             idx_v, yidx_v, yr_v, qt_v, rows_v, stage_v,
             gsem, osem, ysem):
        wid = lax.axis_index("s") * NC + lax.axis_index("c")
        base = wid * BPW

        pltpu.sync_copy(answer_hbm.at[pl.ds(base, BPW)], idx_v)
        pltpu.sync_copy(year_hbm.at[pl.ds(base, BPW)], yidx_v)
        pltpu.sync_copy(qtab_hbm, qt_v)
        pltpu.async_copy(ytab_hbm.at[yidx_v], yr_v, ysem).wait()

        for j in range(NBUF_G):
            pltpu.async_copy(atab_hbm.at[idx_v.at[j]], rows_v.at[j],
                             gsem.at[j])

        def outer(o, carry):
            for j in range(NBUF_G):
                g = o * NBUF_G + j
                oj = j % NBUF_O
                pltpu.make_async_copy(
                    atab_hbm.at[idx_v.at[g]], rows_v.at[j], gsem.at[j]).wait()

                def _wait_out():
                    pltpu.make_async_copy(
                        stage_v.at[oj], out_hbm.at[base], osem.at[oj]).wait()
                if j >= NBUF_O:
                    _wait_out()
                else:
                    pl.when(o > 0)(_wait_out)

                yrow = tuple(yr_v[g, pl.ds(c * L, L)] for c in range(C))

                def qbody(q, ycarry):
                    for c in range(C):
                        stage_v[oj, q, pl.ds(c * L, L)] = (
                            rows_v[j, q, pl.ds(c * L, L)]
                            + qt_v[q, pl.ds(c * L, L)]
                            + ycarry[c])
                    return ycarry

                lax.fori_loop(0, Q, qbody, yrow)

                pltpu.async_copy(stage_v.at[oj], out_hbm.at[base + g],
                                 osem.at[oj])

                def _next_gather():
                    pltpu.async_copy(
                        atab_hbm.at[idx_v.at[g + NBUF_G]], rows_v.at[j],
                        gsem.at[j])
                pl.when(o < BPW // NBUF_G - 1)(_next_gather)
            return carry

        lax.fori_loop(0, BPW // NBUF_G, outer, 0)

        for oj in range(NBUF_O):
            pltpu.make_async_copy(
                stage_v.at[oj], out_hbm.at[base], osem.at[oj]).wait()

    mesh = plsc.VectorSubcoreMesh(core_axis_name="c", subcore_axis_name="s",
                                  num_cores=NC, num_subcores=NS)
    run = pl.kernel(
        body,
        out_type=jax.ShapeDtypeStruct((B, Q, D), jnp.float32),
        mesh=mesh,
        scratch_types=[
            pltpu.VMEM((BPW, Q), jnp.int32),
            pltpu.VMEM((BPW,), jnp.int32),
            pltpu.VMEM((BPW, DP), jnp.float32),
            pltpu.VMEM((Q, D), jnp.float32),
            pltpu.VMEM((NBUF_G, Q, DP), jnp.float32),
            pltpu.VMEM((NBUF_O, Q, D), jnp.float32),
            pltpu.SemaphoreType.DMA((NBUF_G,)),
            pltpu.SemaphoreType.DMA((NBUF_O,)),
            pltpu.SemaphoreType.DMA,
        ],
    )
    return run(answer, year_flat, atab_pad, ytab_pad, question_table)
